# baseline (speedup 1.0000x reference)
import functools

import jax
import jax.numpy as jnp
from jax import lax
from jax.experimental import pallas as pl
from jax.experimental.pallas import tpu as pltpu
from jax.experimental.pallas import tpu_sc as plsc

_B = 16384
_L = 200
_LS = 20
_LSP = 24
_DD, _DP, _DS = 32, 16, 16
_H1, _H2 = 128, 64
_F = 2 * (_DD + _DP + _DS)

_ND, _NP, _NSUB = 13752, 17393, 30
_NC, _NS = 2, 16
_NW = _NC * _NS
_BPW = _B // _NW
_CE = 16
_NCHUNK = _BPW // _CE
_S0, _S1 = 128, 72
_DEPTH = 3


def _features_sc(cd, cp, cs, pd, pp, ps, dis_t, phe_t, sub_t):
    mesh = plsc.VectorSubcoreMesh(core_axis_name="c", subcore_axis_name="s")

    @functools.partial(
        pl.kernel,
        mesh=mesh,
        compiler_params=pltpu.CompilerParams(use_tc_tiling_on_sc=False),
        out_type=jax.ShapeDtypeStruct((_B, _F), jnp.float32),
        scratch_types=[
            pltpu.VMEM((_CE, _L), jnp.int32),
            pltpu.VMEM((_CE, _L), jnp.int32),
            pltpu.VMEM((_CE, _LSP), jnp.int32),
            pltpu.VMEM((_CE, _L), jnp.int32),
            pltpu.VMEM((_CE, _L), jnp.int32),
            pltpu.VMEM((_CE, _LSP), jnp.int32),
            pltpu.VMEM((_DEPTH, _L, _DD), jnp.float32),
            pltpu.VMEM((_DEPTH, _L, _DP), jnp.float32),
            pltpu.VMEM((_DEPTH, _LSP, _DS), jnp.float32),
            pltpu.VMEM((_DEPTH, _L, _DD), jnp.float32),
            pltpu.VMEM((_DEPTH, _L, _DP), jnp.float32),
            pltpu.VMEM((_DEPTH, _LSP, _DS), jnp.float32),
            pltpu.VMEM((_CE, _F), jnp.float32),
            pltpu.VMEM_SHARED((_ND, _DD), jnp.float32),
            pltpu.VMEM_SHARED((_NP, _DP), jnp.float32),
            pltpu.VMEM_SHARED((_NSUB, _DS), jnp.float32),
        ] + [pltpu.SemaphoreType.DMA] * _DEPTH
          + [pltpu.SemaphoreType.DMA],
    )
    def feat_kernel(cd_h, cp_h, cs_h, pd_h, pp_h, ps_h, dis_h, phe_h, sub_h,
                    out_h, cd_i, cp_i, cs_i, pd_i, pp_i, ps_i,
                    cdb, cpb, csb, pdb, ppb, psb, feat,
                    dis_s, phe_s, sub_s, *sems):
        gsems = sems[:_DEPTH]
        slab_sem = sems[_DEPTH]
        wid = lax.axis_index("s") * _NC + lax.axis_index("c")
        base = wid * _BPW

        @pl.when(lax.axis_index("s") == 0)
        def _():
            pltpu.sync_copy(dis_h, dis_s)
            pltpu.sync_copy(phe_h, phe_s)
            pltpu.sync_copy(sub_h, sub_s)

        plsc.subcore_barrier()

        def issue(e, si, sem):
            for idx_i, buf, tbl in ((cd_i, cdb, dis_s), (cp_i, cpb, phe_s),
                                    (pd_i, pdb, dis_s), (pp_i, ppb, phe_s)):
                pltpu.async_copy(tbl.at[idx_i.at[e, pl.ds(0, _S0)]],
                                 buf.at[si, pl.ds(0, _S0)], sem)
                pltpu.async_copy(tbl.at[idx_i.at[e, pl.ds(_S0, _S1)]],
                                 buf.at[si, pl.ds(_S0, _S1)], sem)
            pltpu.async_copy(sub_s.at[cs_i.at[e]], csb.at[si], sem)
            pltpu.async_copy(sub_s.at[ps_i.at[e]], psb.at[si], sem)

        def drain(si, sem):
            pltpu.make_async_copy(dis_h.at[pl.ds(0, _L)], cdb.at[si], sem).wait()
            pltpu.make_async_copy(phe_h.at[pl.ds(0, _L)], cpb.at[si], sem).wait()
            pltpu.make_async_copy(dis_h.at[pl.ds(0, _L)], pdb.at[si], sem).wait()
            pltpu.make_async_copy(phe_h.at[pl.ds(0, _L)], ppb.at[si], sem).wait()
            pltpu.make_async_copy(sub_h.at[pl.ds(0, _LSP)], csb.at[si], sem).wait()
            pltpu.make_async_copy(sub_h.at[pl.ds(0, _LSP)], psb.at[si], sem).wait()

        def reduce_sub(buf, si, e, col):
            accs = [jnp.zeros((16,), jnp.float32) for _ in range(4)]
            for j in range(_LS):
                accs[j % 4] = accs[j % 4] + buf[si, j, pl.ds(0, 16)]
            tot = (accs[0] + accs[1]) + (accs[2] + accs[3])
            feat[e, pl.ds(col, 16)] = tot * (1.0 / _LS)

        def reduce_all(si, e):
            nrows = 4
            z = jnp.zeros((16,), jnp.float32)

            @plsc.parallel_loop(0, _L, nrows, unroll=2,
                                carry=tuple(z for _ in range(12)))
            def accs(r0, accs):
                accs = list(accs)
                for i in range(nrows):
                    row = r0 + i
                    ch = i % 2
                    accs[0 + ch] = accs[0 + ch] + cdb[si, row, pl.ds(0, 16)]
                    accs[2 + ch] = accs[2 + ch] + cdb[si, row, pl.ds(16, 16)]
                    accs[4 + ch] = accs[4 + ch] + cpb[si, row, pl.ds(0, 16)]
                    accs[6 + ch] = accs[6 + ch] + pdb[si, row, pl.ds(0, 16)]
                    accs[8 + ch] = accs[8 + ch] + pdb[si, row, pl.ds(16, 16)]
                    accs[10 + ch] = accs[10 + ch] + ppb[si, row, pl.ds(0, 16)]
                return tuple(accs)

            s = 1.0 / _L
            feat[e, pl.ds(0, 16)] = (accs[0] + accs[1]) * s
            feat[e, pl.ds(16, 16)] = (accs[2] + accs[3]) * s
            feat[e, pl.ds(_DD, 16)] = (accs[4] + accs[5]) * s
            feat[e, pl.ds(64, 16)] = (accs[6] + accs[7]) * s
            feat[e, pl.ds(80, 16)] = (accs[8] + accs[9]) * s
            feat[e, pl.ds(64 + _DD, 16)] = (accs[10] + accs[11]) * s
            reduce_sub(csb, si, e, _DD + _DP)
            reduce_sub(psb, si, e, 64 + _DD + _DP)

        def chunk_body(c, _):
            cbase = base + c * _CE
            scp = []
            for src, dst in ((cd_h, cd_i), (cp_h, cp_i), (cs_h, cs_i),
                             (pd_h, pd_i), (pp_h, pp_i), (ps_h, ps_i)):
                scp.append(pltpu.async_copy(
                    src.at[pl.ds(cbase, _CE)], dst, slab_sem))
            for d in scp:
                d.wait()

            for d in range(_DEPTH - 1):
                issue(d, d, gsems[d])

            def group_body(k, __):
                e = k * _DEPTH
                for d in range(_DEPTH):
                    ed = e + d
                    nxt = ed + _DEPTH - 1
                    tgt = (d + _DEPTH - 1) % _DEPTH
                    if d == 0:
                        issue(nxt, tgt, gsems[tgt])
                    else:
                        @pl.when(nxt < _CE)
                        def _(nxt=nxt, tgt=tgt):
                            issue(nxt, tgt, gsems[tgt])
                    drain(d, gsems[d])
                    reduce_all(d, ed)
                return 0

            lax.fori_loop(0, _CE // _DEPTH, group_body, 0)
            for t in range((_CE // _DEPTH) * _DEPTH, _CE):
                st = t % _DEPTH
                drain(st, gsems[st])
                reduce_all(st, t)
            pltpu.sync_copy(feat, out_h.at[pl.ds(cbase, _CE)])
            return 0

        lax.fori_loop(0, _NCHUNK, chunk_body, 0)

    return feat_kernel(cd, cp, cs, pd, pp, ps, dis_t, phe_t, sub_t)


def _mlp_tc(feat, w1, b1, w2, b2, w3t, b3):
    blk = 1024

    def body(x_ref, w1_ref, b1_ref, w2_ref, b2_ref, w3t_ref, b3_ref, o_ref):
        x = x_ref[...]
        h = jnp.dot(x, w1_ref[...], preferred_element_type=jnp.float32)
        h = h + b1_ref[...]
        h = jnp.where(h > 0, h, h * 0.01)
        h = jnp.dot(h, w2_ref[...], preferred_element_type=jnp.float32)
        h = h + b2_ref[...]
        h = jnp.where(h > 0, h, h * 0.01)
        o = jnp.sum(h * w3t_ref[...], axis=1, keepdims=True) + b3_ref[...]
        o_ref[...] = o

    return pl.pallas_call(
        body,
        grid=(_B // blk,),
        in_specs=[
            pl.BlockSpec((blk, _F), lambda i: (i, 0)),
            pl.BlockSpec((_F, _H1), lambda i: (0, 0)),
            pl.BlockSpec((1, _H1), lambda i: (0, 0)),
            pl.BlockSpec((_H1, _H2), lambda i: (0, 0)),
            pl.BlockSpec((1, _H2), lambda i: (0, 0)),
            pl.BlockSpec((1, _H2), lambda i: (0, 0)),
            pl.BlockSpec((1, 1), lambda i: (0, 0)),
        ],
        out_specs=pl.BlockSpec((blk, 1), lambda i: (i, 0)),
        out_shape=jax.ShapeDtypeStruct((_B, 1), jnp.float32),
    )(feat, w1, b1, w2, b2, w3t, b3)


def kernel(compound_diseases, compound_phenotypes,
           compound_subcellular_locations, protein_diseases,
           protein_phenotypes, protein_subcellular_locations,
           disease_table, phenotype_table, sub_table,
           W1, b1, W2, b2, W3, b3):
    cd = compound_diseases.astype(jnp.int32)
    cp = compound_phenotypes.astype(jnp.int32)
    pd = protein_diseases.astype(jnp.int32)
    pp = protein_phenotypes.astype(jnp.int32)
    cs = jnp.pad(compound_subcellular_locations.astype(jnp.int32),
                 ((0, 0), (0, _LSP - _LS)))
    ps = jnp.pad(protein_subcellular_locations.astype(jnp.int32),
                 ((0, 0), (0, _LSP - _LS)))

    feat = _features_sc(cd, cp, cs, pd, pp, ps,
                        disease_table, phenotype_table, sub_table)
    return _mlp_tc(feat, W1, b1.reshape(1, _H1), W2, b2.reshape(1, _H2),
                   W3.reshape(1, _H2), b3.reshape(1, 1))


# speedup vs baseline: 1.0630x; 1.0630x over previous
import functools

import jax
import jax.numpy as jnp
from jax import lax
from jax.experimental import pallas as pl
from jax.experimental.pallas import tpu as pltpu
from jax.experimental.pallas import tpu_sc as plsc

_B = 16384
_L = 200
_LS = 20
_LSP = 24
_LS2 = 2 * _LSP
_DD, _DP, _DS = 32, 16, 16
_H1, _H2 = 128, 64
_F = 2 * (_DD + _DP + _DS)

_ND, _NP, _NSUB = 13752, 17393, 30
_NC, _NS = 2, 16
_NW = _NC * _NS
_BPW = _B // _NW
_CE = 16
_NCHUNK = _BPW // _CE
_S0, _S1 = 128, 72
_DEPTH = 2
_NSS = (_CE * _LS2) // 128


def _features_sc(cd, cp, sidx, pd, pp, dis_t, phe_t, sub_t):
    mesh = plsc.VectorSubcoreMesh(core_axis_name="c", subcore_axis_name="s")

    @functools.partial(
        pl.kernel,
        mesh=mesh,
        compiler_params=pltpu.CompilerParams(use_tc_tiling_on_sc=False),
        out_type=jax.ShapeDtypeStruct((_B, _F), jnp.float32),
        scratch_types=[
            pltpu.VMEM((_CE, _L), jnp.int32),
            pltpu.VMEM((_CE, _L), jnp.int32),
            pltpu.VMEM((_CE * _LS2,), jnp.int32),
            pltpu.VMEM((_CE, _L), jnp.int32),
            pltpu.VMEM((_CE, _L), jnp.int32),
            pltpu.VMEM((_DEPTH, _L, _DD), jnp.float32),
            pltpu.VMEM((_DEPTH, _L, _DP), jnp.float32),
            pltpu.VMEM((_DEPTH, _L, _DD), jnp.float32),
            pltpu.VMEM((_DEPTH, _L, _DP), jnp.float32),
            pltpu.VMEM((_CE * _LS2, _DS), jnp.float32),
            pltpu.VMEM((_CE, _F), jnp.float32),
            pltpu.VMEM_SHARED((_ND, _DD), jnp.float32),
            pltpu.VMEM_SHARED((_NP, _DP), jnp.float32),
            pltpu.VMEM_SHARED((_NSUB, _DS), jnp.float32),
        ] + [pltpu.SemaphoreType.DMA] * _DEPTH
          + [pltpu.SemaphoreType.DMA],
    )
    def feat_kernel(cd_h, cp_h, sidx_h, pd_h, pp_h, dis_h, phe_h, sub_h,
                    out_h, cd_i, cp_i, sidx_i, pd_i, pp_i,
                    cdb, cpb, pdb, ppb, subr, feat,
                    dis_s, phe_s, sub_s, *sems):
        gsems = sems[:_DEPTH]
        slab_sem = sems[_DEPTH]
        wid = lax.axis_index("s") * _NC + lax.axis_index("c")
        base = wid * _BPW

        @pl.when(lax.axis_index("s") == 0)
        def _():
            pltpu.sync_copy(dis_h, dis_s)
            pltpu.sync_copy(phe_h, phe_s)
            pltpu.sync_copy(sub_h, sub_s)

        plsc.subcore_barrier()

        def issue(e, si, sem):
            for idx_i, buf, tbl in ((cd_i, cdb, dis_s), (cp_i, cpb, phe_s),
                                    (pd_i, pdb, dis_s), (pp_i, ppb, phe_s)):
                pltpu.async_copy(tbl.at[idx_i.at[e, pl.ds(0, _S0)]],
                                 buf.at[si, pl.ds(0, _S0)], sem)
                pltpu.async_copy(tbl.at[idx_i.at[e, pl.ds(_S0, _S1)]],
                                 buf.at[si, pl.ds(_S0, _S1)], sem)

        def drain(si, sem):
            pltpu.make_async_copy(dis_h.at[pl.ds(0, _L)], cdb.at[si], sem).wait()
            pltpu.make_async_copy(phe_h.at[pl.ds(0, _L)], cpb.at[si], sem).wait()
            pltpu.make_async_copy(dis_h.at[pl.ds(0, _L)], pdb.at[si], sem).wait()
            pltpu.make_async_copy(phe_h.at[pl.ds(0, _L)], ppb.at[si], sem).wait()

        def reduce_sub(e, off, col):
            accs = [jnp.zeros((16,), jnp.float32) for _ in range(4)]
            rbase = e * _LS2 + off
            for j in range(_LS):
                accs[j % 4] = accs[j % 4] + subr[rbase + j, pl.ds(0, 16)]
            tot = (accs[0] + accs[1]) + (accs[2] + accs[3])
            feat[e, pl.ds(col, 16)] = tot * (1.0 / _LS)

        def reduce_all(si, e):
            nrows = 4
            z = jnp.zeros((16,), jnp.float32)

            @plsc.parallel_loop(0, _L, nrows, unroll=2,
                                carry=tuple(z for _ in range(12)))
            def accs(r0, accs):
                accs = list(accs)
                for i in range(nrows):
                    row = r0 + i
                    ch = i % 2
                    accs[0 + ch] = accs[0 + ch] + cdb[si, row, pl.ds(0, 16)]
                    accs[2 + ch] = accs[2 + ch] + cdb[si, row, pl.ds(16, 16)]
                    accs[4 + ch] = accs[4 + ch] + cpb[si, row, pl.ds(0, 16)]
                    accs[6 + ch] = accs[6 + ch] + pdb[si, row, pl.ds(0, 16)]
                    accs[8 + ch] = accs[8 + ch] + pdb[si, row, pl.ds(16, 16)]
                    accs[10 + ch] = accs[10 + ch] + ppb[si, row, pl.ds(0, 16)]
                return tuple(accs)

            s = 1.0 / _L
            feat[e, pl.ds(0, 16)] = (accs[0] + accs[1]) * s
            feat[e, pl.ds(16, 16)] = (accs[2] + accs[3]) * s
            feat[e, pl.ds(_DD, 16)] = (accs[4] + accs[5]) * s
            feat[e, pl.ds(64, 16)] = (accs[6] + accs[7]) * s
            feat[e, pl.ds(80, 16)] = (accs[8] + accs[9]) * s
            feat[e, pl.ds(64 + _DD, 16)] = (accs[10] + accs[11]) * s
            reduce_sub(e, 0, _DD + _DP)
            reduce_sub(e, _LSP, 64 + _DD + _DP)

        def chunk_body(c, _):
            cbase = base + c * _CE
            scp = []
            for src, dst in ((cd_h, cd_i), (cp_h, cp_i), (pd_h, pd_i),
                             (pp_h, pp_i)):
                scp.append(pltpu.async_copy(
                    src.at[pl.ds(cbase, _CE)], dst, slab_sem))
            scp.append(pltpu.async_copy(
                sidx_h.at[pl.ds(cbase * _LS2, _CE * _LS2)], sidx_i, slab_sem))
            for d in scp:
                d.wait()

            for k in range(_NSS):
                pltpu.async_copy(
                    sub_s.at[sidx_i.at[pl.ds(k * 128, 128)]],
                    subr.at[pl.ds(k * 128, 128)], slab_sem)

            for d in range(_DEPTH - 1):
                issue(d, d, gsems[d])

            pltpu.make_async_copy(phe_h.at[pl.ds(0, _CE * _LS2)], subr,
                                  slab_sem).wait()

            def group_body(k, __):
                e = k * _DEPTH
                for d in range(_DEPTH):
                    ed = e + d
                    nxt = ed + _DEPTH - 1
                    tgt = (d + _DEPTH - 1) % _DEPTH
                    if d == 0:
                        issue(nxt, tgt, gsems[tgt])
                    else:
                        @pl.when(nxt < _CE)
                        def _(nxt=nxt, tgt=tgt):
                            issue(nxt, tgt, gsems[tgt])
                    drain(d, gsems[d])
                    reduce_all(d, ed)
                return 0

            lax.fori_loop(0, _CE // _DEPTH, group_body, 0)
            for t in range((_CE // _DEPTH) * _DEPTH, _CE):
                st = t % _DEPTH
                drain(st, gsems[st])
                reduce_all(st, t)
            pltpu.sync_copy(feat, out_h.at[pl.ds(cbase, _CE)])
            return 0

        lax.fori_loop(0, _NCHUNK, chunk_body, 0)

    return feat_kernel(cd, cp, sidx, pd, pp, dis_t, phe_t, sub_t)


def _mlp_tc(feat, w1, b1, w2, b2, w3t, b3):
    blk = 1024

    def body(x_ref, w1_ref, b1_ref, w2_ref, b2_ref, w3t_ref, b3_ref, o_ref):
        x = x_ref[...]
        h = jnp.dot(x, w1_ref[...], preferred_element_type=jnp.float32)
        h = h + b1_ref[...]
        h = jnp.where(h > 0, h, h * 0.01)
        h = jnp.dot(h, w2_ref[...], preferred_element_type=jnp.float32)
        h = h + b2_ref[...]
        h = jnp.where(h > 0, h, h * 0.01)
        o = jnp.sum(h * w3t_ref[...], axis=1, keepdims=True) + b3_ref[...]
        o_ref[...] = o

    return pl.pallas_call(
        body,
        grid=(_B // blk,),
        in_specs=[
            pl.BlockSpec((blk, _F), lambda i: (i, 0)),
            pl.BlockSpec((_F, _H1), lambda i: (0, 0)),
            pl.BlockSpec((1, _H1), lambda i: (0, 0)),
            pl.BlockSpec((_H1, _H2), lambda i: (0, 0)),
            pl.BlockSpec((1, _H2), lambda i: (0, 0)),
            pl.BlockSpec((1, _H2), lambda i: (0, 0)),
            pl.BlockSpec((1, 1), lambda i: (0, 0)),
        ],
        out_specs=pl.BlockSpec((blk, 1), lambda i: (i, 0)),
        out_shape=jax.ShapeDtypeStruct((_B, 1), jnp.float32),
    )(feat, w1, b1, w2, b2, w3t, b3)


def kernel(compound_diseases, compound_phenotypes,
           compound_subcellular_locations, protein_diseases,
           protein_phenotypes, protein_subcellular_locations,
           disease_table, phenotype_table, sub_table,
           W1, b1, W2, b2, W3, b3):
    cd = compound_diseases.astype(jnp.int32)
    cp = compound_phenotypes.astype(jnp.int32)
    pd = protein_diseases.astype(jnp.int32)
    pp = protein_phenotypes.astype(jnp.int32)
    cs = jnp.pad(compound_subcellular_locations.astype(jnp.int32),
                 ((0, 0), (0, _LSP - _LS)))
    ps = jnp.pad(protein_subcellular_locations.astype(jnp.int32),
                 ((0, 0), (0, _LSP - _LS)))
    sidx = jnp.concatenate([cs, ps], axis=1).reshape(-1)

    feat = _features_sc(cd, cp, sidx, pd, pp,
                        disease_table, phenotype_table, sub_table)
    return _mlp_tc(feat, W1, b1.reshape(1, _H1), W2, b2.reshape(1, _H2),
                   W3.reshape(1, _H2), b3.reshape(1, 1))


# speedup vs baseline: 1.0677x; 1.0044x over previous
import functools

import jax
import jax.numpy as jnp
from jax import lax
from jax.experimental import pallas as pl
from jax.experimental.pallas import tpu as pltpu
from jax.experimental.pallas import tpu_sc as plsc

_B = 16384
_L = 200
_LS = 20
_LSP = 24
_LS2 = 2 * _LSP
_DD, _DP, _DS = 32, 16, 16
_H1, _H2 = 128, 64
_F = 2 * (_DD + _DP + _DS)

_ND, _NP, _NSUB = 13752, 17393, 30
_NC, _NS = 2, 16
_NW = _NC * _NS
_BPW = _B // _NW
_CE = 16
_NCHUNK = _BPW // _CE
_S0, _S1 = 128, 72
_DEPTH = 3
_NSS = (_CE * _LS2) // 128


def _features_sc(cd, cp, sidx, pd, pp, dis_t, phe_t, sub_t):
    mesh = plsc.VectorSubcoreMesh(core_axis_name="c", subcore_axis_name="s")

    @functools.partial(
        pl.kernel,
        mesh=mesh,
        compiler_params=pltpu.CompilerParams(use_tc_tiling_on_sc=False),
        out_type=jax.ShapeDtypeStruct((_B, _F), jnp.float32),
        scratch_types=[
            pltpu.VMEM((_CE, _L), jnp.int32),
            pltpu.VMEM((_CE, _L), jnp.int32),
            pltpu.VMEM((_CE * _LS2,), jnp.int32),
            pltpu.VMEM((_CE, _L), jnp.int32),
            pltpu.VMEM((_CE, _L), jnp.int32),
            pltpu.VMEM((_DEPTH, _L, _DD), jnp.float32),
            pltpu.VMEM((_DEPTH, _L, _DP), jnp.float32),
            pltpu.VMEM((_DEPTH, _L, _DD), jnp.float32),
            pltpu.VMEM((_DEPTH, _L, _DP), jnp.float32),
            pltpu.VMEM((_CE * _LS2, _DS), jnp.float32),
            pltpu.VMEM((_CE, _F), jnp.float32),
            pltpu.VMEM_SHARED((_ND, _DD), jnp.float32),
            pltpu.VMEM_SHARED((_NP, _DP), jnp.float32),
            pltpu.VMEM_SHARED((_NSUB, _DS), jnp.float32),
        ] + [pltpu.SemaphoreType.DMA] * _DEPTH
          + [pltpu.SemaphoreType.DMA],
    )
    def feat_kernel(cd_h, cp_h, sidx_h, pd_h, pp_h, dis_h, phe_h, sub_h,
                    out_h, cd_i, cp_i, sidx_i, pd_i, pp_i,
                    cdb, cpb, pdb, ppb, subr, feat,
                    dis_s, phe_s, sub_s, *sems):
        gsems = sems[:_DEPTH]
        slab_sem = sems[_DEPTH]
        wid = lax.axis_index("s") * _NC + lax.axis_index("c")
        base = wid * _BPW

        @pl.when(lax.axis_index("s") == 0)
        def _():
            pltpu.sync_copy(dis_h, dis_s)
            pltpu.sync_copy(phe_h, phe_s)
            pltpu.sync_copy(sub_h, sub_s)

        plsc.subcore_barrier()

        def issue(e, si, sem):
            for idx_i, buf, tbl in ((cd_i, cdb, dis_s), (cp_i, cpb, phe_s),
                                    (pd_i, pdb, dis_s), (pp_i, ppb, phe_s)):
                pltpu.async_copy(tbl.at[idx_i.at[e, pl.ds(0, _S0)]],
                                 buf.at[si, pl.ds(0, _S0)], sem)
                pltpu.async_copy(tbl.at[idx_i.at[e, pl.ds(_S0, _S1)]],
                                 buf.at[si, pl.ds(_S0, _S1)], sem)

        def drain(si, sem):
            pltpu.make_async_copy(dis_h.at[pl.ds(0, _L)], cdb.at[si], sem).wait()
            pltpu.make_async_copy(phe_h.at[pl.ds(0, _L)], cpb.at[si], sem).wait()
            pltpu.make_async_copy(dis_h.at[pl.ds(0, _L)], pdb.at[si], sem).wait()
            pltpu.make_async_copy(phe_h.at[pl.ds(0, _L)], ppb.at[si], sem).wait()

        def reduce_sub(e, off, col):
            accs = [jnp.zeros((16,), jnp.float32) for _ in range(4)]
            rbase = e * _LS2 + off
            for j in range(_LS):
                accs[j % 4] = accs[j % 4] + subr[rbase + j, pl.ds(0, 16)]
            tot = (accs[0] + accs[1]) + (accs[2] + accs[3])
            feat[e, pl.ds(col, 16)] = tot * (1.0 / _LS)

        def reduce_all(si, e):
            nrows = 4
            z = jnp.zeros((16,), jnp.float32)

            @plsc.parallel_loop(0, _L, nrows, unroll=2,
                                carry=tuple(z for _ in range(12)))
            def accs(r0, accs):
                accs = list(accs)
                for i in range(nrows):
                    row = r0 + i
                    ch = i % 2
                    accs[0 + ch] = accs[0 + ch] + cdb[si, row, pl.ds(0, 16)]
                    accs[2 + ch] = accs[2 + ch] + cdb[si, row, pl.ds(16, 16)]
                    accs[4 + ch] = accs[4 + ch] + cpb[si, row, pl.ds(0, 16)]
                    accs[6 + ch] = accs[6 + ch] + pdb[si, row, pl.ds(0, 16)]
                    accs[8 + ch] = accs[8 + ch] + pdb[si, row, pl.ds(16, 16)]
                    accs[10 + ch] = accs[10 + ch] + ppb[si, row, pl.ds(0, 16)]
                return tuple(accs)

            s = 1.0 / _L
            feat[e, pl.ds(0, 16)] = (accs[0] + accs[1]) * s
            feat[e, pl.ds(16, 16)] = (accs[2] + accs[3]) * s
            feat[e, pl.ds(_DD, 16)] = (accs[4] + accs[5]) * s
            feat[e, pl.ds(64, 16)] = (accs[6] + accs[7]) * s
            feat[e, pl.ds(80, 16)] = (accs[8] + accs[9]) * s
            feat[e, pl.ds(64 + _DD, 16)] = (accs[10] + accs[11]) * s
            reduce_sub(e, 0, _DD + _DP)
            reduce_sub(e, _LSP, 64 + _DD + _DP)

        def chunk_body(c, _):
            cbase = base + c * _CE
            scp = []
            for src, dst in ((cd_h, cd_i), (cp_h, cp_i), (pd_h, pd_i),
                             (pp_h, pp_i)):
                scp.append(pltpu.async_copy(
                    src.at[pl.ds(cbase, _CE)], dst, slab_sem))
            scp.append(pltpu.async_copy(
                sidx_h.at[pl.ds(cbase * _LS2, _CE * _LS2)], sidx_i, slab_sem))
            for d in scp:
                d.wait()

            for k in range(_NSS):
                pltpu.async_copy(
                    sub_s.at[sidx_i.at[pl.ds(k * 128, 128)]],
                    subr.at[pl.ds(k * 128, 128)], slab_sem)

            for d in range(_DEPTH - 1):
                issue(d, d, gsems[d])

            pltpu.make_async_copy(phe_h.at[pl.ds(0, _CE * _LS2)], subr,
                                  slab_sem).wait()

            def group_body(k, __):
                e = k * _DEPTH
                for d in range(_DEPTH):
                    ed = e + d
                    nxt = ed + _DEPTH - 1
                    tgt = (d + _DEPTH - 1) % _DEPTH
                    if d == 0:
                        issue(nxt, tgt, gsems[tgt])
                    else:
                        @pl.when(nxt < _CE)
                        def _(nxt=nxt, tgt=tgt):
                            issue(nxt, tgt, gsems[tgt])
                    drain(d, gsems[d])
                    reduce_all(d, ed)
                return 0

            lax.fori_loop(0, _CE // _DEPTH, group_body, 0)
            for t in range((_CE // _DEPTH) * _DEPTH, _CE):
                st = t % _DEPTH
                drain(st, gsems[st])
                reduce_all(st, t)
            pltpu.sync_copy(feat, out_h.at[pl.ds(cbase, _CE)])
            return 0

        lax.fori_loop(0, _NCHUNK, chunk_body, 0)

    return feat_kernel(cd, cp, sidx, pd, pp, dis_t, phe_t, sub_t)


def _mlp_tc(feat, w1, b1, w2, b2, w3t, b3):
    blk = 1024

    def body(x_ref, w1_ref, b1_ref, w2_ref, b2_ref, w3t_ref, b3_ref, o_ref):
        x = x_ref[...]
        h = jnp.dot(x, w1_ref[...], preferred_element_type=jnp.float32)
        h = h + b1_ref[...]
        h = jnp.where(h > 0, h, h * 0.01)
        h = jnp.dot(h, w2_ref[...], preferred_element_type=jnp.float32)
        h = h + b2_ref[...]
        h = jnp.where(h > 0, h, h * 0.01)
        o = jnp.sum(h * w3t_ref[...], axis=1, keepdims=True) + b3_ref[...]
        o_ref[...] = o

    return pl.pallas_call(
        body,
        grid=(_B // blk,),
        in_specs=[
            pl.BlockSpec((blk, _F), lambda i: (i, 0)),
            pl.BlockSpec((_F, _H1), lambda i: (0, 0)),
            pl.BlockSpec((1, _H1), lambda i: (0, 0)),
            pl.BlockSpec((_H1, _H2), lambda i: (0, 0)),
            pl.BlockSpec((1, _H2), lambda i: (0, 0)),
            pl.BlockSpec((1, _H2), lambda i: (0, 0)),
            pl.BlockSpec((1, 1), lambda i: (0, 0)),
        ],
        out_specs=pl.BlockSpec((blk, 1), lambda i: (i, 0)),
        out_shape=jax.ShapeDtypeStruct((_B, 1), jnp.float32),
    )(feat, w1, b1, w2, b2, w3t, b3)


def kernel(compound_diseases, compound_phenotypes,
           compound_subcellular_locations, protein_diseases,
           protein_phenotypes, protein_subcellular_locations,
           disease_table, phenotype_table, sub_table,
           W1, b1, W2, b2, W3, b3):
    cd = compound_diseases.astype(jnp.int32)
    cp = compound_phenotypes.astype(jnp.int32)
    pd = protein_diseases.astype(jnp.int32)
    pp = protein_phenotypes.astype(jnp.int32)
    cs = jnp.pad(compound_subcellular_locations.astype(jnp.int32),
                 ((0, 0), (0, _LSP - _LS)))
    ps = jnp.pad(protein_subcellular_locations.astype(jnp.int32),
                 ((0, 0), (0, _LSP - _LS)))
    sidx = jnp.concatenate([cs, ps], axis=1).reshape(-1)

    feat = _features_sc(cd, cp, sidx, pd, pp,
                        disease_table, phenotype_table, sub_table)
    return _mlp_tc(feat, W1, b1.reshape(1, _H1), W2, b2.reshape(1, _H2),
                   W3.reshape(1, _H2), b3.reshape(1, 1))


# speedup vs baseline: 1.0706x; 1.0027x over previous
import functools

import jax
import jax.numpy as jnp
from jax import lax
from jax.experimental import pallas as pl
from jax.experimental.pallas import tpu as pltpu
from jax.experimental.pallas import tpu_sc as plsc

_B = 16384
_L = 200
_LS = 20
_LSP = 24
_LS2 = 2 * _LSP
_DD, _DP, _DS = 32, 16, 16
_H1, _H2 = 128, 64
_F = 2 * (_DD + _DP + _DS)

_ND, _NP, _NSUB = 13752, 17393, 30
_NC, _NS = 2, 16
_NW = _NC * _NS
_BPW = _B // _NW
_CE = 16
_NCHUNK = _BPW // _CE
_S0, _S1 = 128, 72
_DEPTH = 3
_NSS = (_CE * _LS2) // 128


def _features_sc(cd, cp, sidx, pd, pp, dis_t, phe_t, sub_t):
    mesh = plsc.VectorSubcoreMesh(core_axis_name="c", subcore_axis_name="s")

    @functools.partial(
        pl.kernel,
        mesh=mesh,
        compiler_params=pltpu.CompilerParams(use_tc_tiling_on_sc=False),
        out_type=jax.ShapeDtypeStruct((_B, _F), jnp.float32),
        scratch_types=[
            pltpu.VMEM((_CE, _L), jnp.int32),
            pltpu.VMEM((_CE, _L), jnp.int32),
            pltpu.VMEM((_CE * _LS2,), jnp.int32),
            pltpu.VMEM((_CE, _L), jnp.int32),
            pltpu.VMEM((_CE, _L), jnp.int32),
            pltpu.VMEM((_DEPTH, _L, _DD), jnp.float32),
            pltpu.VMEM((_DEPTH, _L, _DP), jnp.float32),
            pltpu.VMEM((_DEPTH, _L, _DD), jnp.float32),
            pltpu.VMEM((_DEPTH, _L, _DP), jnp.float32),
            pltpu.VMEM((_CE * _LS2, _DS), jnp.float32),
            pltpu.VMEM((_CE, _F), jnp.float32),
            pltpu.VMEM_SHARED((_ND, _DD), jnp.float32),
            pltpu.VMEM_SHARED((_NP, _DP), jnp.float32),
            pltpu.VMEM_SHARED((_NSUB, _DS), jnp.float32),
        ] + [pltpu.SemaphoreType.DMA] * _DEPTH
          + [pltpu.SemaphoreType.DMA],
    )
    def feat_kernel(cd_h, cp_h, sidx_h, pd_h, pp_h, dis_h, phe_h, sub_h,
                    out_h, cd_i, cp_i, sidx_i, pd_i, pp_i,
                    cdb, cpb, pdb, ppb, subr, feat,
                    dis_s, phe_s, sub_s, *sems):
        gsems = sems[:_DEPTH]
        slab_sem = sems[_DEPTH]
        wid = lax.axis_index("s") * _NC + lax.axis_index("c")
        base = wid * _BPW

        @pl.when(lax.axis_index("s") == 0)
        def _():
            pltpu.sync_copy(dis_h, dis_s)
            pltpu.sync_copy(phe_h, phe_s)
            pltpu.sync_copy(sub_h, sub_s)

        plsc.subcore_barrier()

        def issue(e, si, sem):
            for idx_i, buf, tbl in ((cd_i, cdb, dis_s), (cp_i, cpb, phe_s),
                                    (pd_i, pdb, dis_s), (pp_i, ppb, phe_s)):
                pltpu.async_copy(tbl.at[idx_i.at[e]], buf.at[si], sem)

        def drain(si, sem):
            pltpu.make_async_copy(dis_h.at[pl.ds(0, _L)], cdb.at[si], sem).wait()
            pltpu.make_async_copy(phe_h.at[pl.ds(0, _L)], cpb.at[si], sem).wait()
            pltpu.make_async_copy(dis_h.at[pl.ds(0, _L)], pdb.at[si], sem).wait()
            pltpu.make_async_copy(phe_h.at[pl.ds(0, _L)], ppb.at[si], sem).wait()

        def reduce_sub(e, off, col):
            accs = [jnp.zeros((16,), jnp.float32) for _ in range(4)]
            rbase = e * _LS2 + off
            for j in range(_LS):
                accs[j % 4] = accs[j % 4] + subr[rbase + j, pl.ds(0, 16)]
            tot = (accs[0] + accs[1]) + (accs[2] + accs[3])
            feat[e, pl.ds(col, 16)] = tot * (1.0 / _LS)

        def reduce_all(si, e):
            nrows = 4
            z = jnp.zeros((16,), jnp.float32)

            @plsc.parallel_loop(0, _L, nrows, unroll=2,
                                carry=tuple(z for _ in range(12)))
            def accs(r0, accs):
                accs = list(accs)
                for i in range(nrows):
                    row = r0 + i
                    ch = i % 2
                    accs[0 + ch] = accs[0 + ch] + cdb[si, row, pl.ds(0, 16)]
                    accs[2 + ch] = accs[2 + ch] + cdb[si, row, pl.ds(16, 16)]
                    accs[4 + ch] = accs[4 + ch] + cpb[si, row, pl.ds(0, 16)]
                    accs[6 + ch] = accs[6 + ch] + pdb[si, row, pl.ds(0, 16)]
                    accs[8 + ch] = accs[8 + ch] + pdb[si, row, pl.ds(16, 16)]
                    accs[10 + ch] = accs[10 + ch] + ppb[si, row, pl.ds(0, 16)]
                return tuple(accs)

            s = 1.0 / _L
            feat[e, pl.ds(0, 16)] = (accs[0] + accs[1]) * s
            feat[e, pl.ds(16, 16)] = (accs[2] + accs[3]) * s
            feat[e, pl.ds(_DD, 16)] = (accs[4] + accs[5]) * s
            feat[e, pl.ds(64, 16)] = (accs[6] + accs[7]) * s
            feat[e, pl.ds(80, 16)] = (accs[8] + accs[9]) * s
            feat[e, pl.ds(64 + _DD, 16)] = (accs[10] + accs[11]) * s
            reduce_sub(e, 0, _DD + _DP)
            reduce_sub(e, _LSP, 64 + _DD + _DP)

        def chunk_body(c, _):
            cbase = base + c * _CE
            scp = []
            for src, dst in ((cd_h, cd_i), (cp_h, cp_i), (pd_h, pd_i),
                             (pp_h, pp_i)):
                scp.append(pltpu.async_copy(
                    src.at[pl.ds(cbase, _CE)], dst, slab_sem))
            scp.append(pltpu.async_copy(
                sidx_h.at[pl.ds(cbase * _LS2, _CE * _LS2)], sidx_i, slab_sem))
            for d in scp:
                d.wait()

            for k in range(_NSS):
                pltpu.async_copy(
                    sub_s.at[sidx_i.at[pl.ds(k * 128, 128)]],
                    subr.at[pl.ds(k * 128, 128)], slab_sem)

            for d in range(_DEPTH - 1):
                issue(d, d, gsems[d])

            pltpu.make_async_copy(phe_h.at[pl.ds(0, _CE * _LS2)], subr,
                                  slab_sem).wait()

            def group_body(k, __):
                e = k * _DEPTH
                for d in range(_DEPTH):
                    ed = e + d
                    nxt = ed + _DEPTH - 1
                    tgt = (d + _DEPTH - 1) % _DEPTH
                    if d == 0:
                        issue(nxt, tgt, gsems[tgt])
                    else:
                        @pl.when(nxt < _CE)
                        def _(nxt=nxt, tgt=tgt):
                            issue(nxt, tgt, gsems[tgt])
                    drain(d, gsems[d])
                    reduce_all(d, ed)
                return 0

            lax.fori_loop(0, _CE // _DEPTH, group_body, 0)
            for t in range((_CE // _DEPTH) * _DEPTH, _CE):
                st = t % _DEPTH
                drain(st, gsems[st])
                reduce_all(st, t)
            pltpu.sync_copy(feat, out_h.at[pl.ds(cbase, _CE)])
            return 0

        lax.fori_loop(0, _NCHUNK, chunk_body, 0)

    return feat_kernel(cd, cp, sidx, pd, pp, dis_t, phe_t, sub_t)


def _mlp_tc(feat, w1, b1, w2, b2, w3t, b3):
    blk = 1024

    def body(x_ref, w1_ref, b1_ref, w2_ref, b2_ref, w3t_ref, b3_ref, o_ref):
        x = x_ref[...]
        h = jnp.dot(x, w1_ref[...], preferred_element_type=jnp.float32)
        h = h + b1_ref[...]
        h = jnp.where(h > 0, h, h * 0.01)
        h = jnp.dot(h, w2_ref[...], preferred_element_type=jnp.float32)
        h = h + b2_ref[...]
        h = jnp.where(h > 0, h, h * 0.01)
        o = jnp.sum(h * w3t_ref[...], axis=1, keepdims=True) + b3_ref[...]
        o_ref[...] = o

    return pl.pallas_call(
        body,
        grid=(_B // blk,),
        in_specs=[
            pl.BlockSpec((blk, _F), lambda i: (i, 0)),
            pl.BlockSpec((_F, _H1), lambda i: (0, 0)),
            pl.BlockSpec((1, _H1), lambda i: (0, 0)),
            pl.BlockSpec((_H1, _H2), lambda i: (0, 0)),
            pl.BlockSpec((1, _H2), lambda i: (0, 0)),
            pl.BlockSpec((1, _H2), lambda i: (0, 0)),
            pl.BlockSpec((1, 1), lambda i: (0, 0)),
        ],
        out_specs=pl.BlockSpec((blk, 1), lambda i: (i, 0)),
        out_shape=jax.ShapeDtypeStruct((_B, 1), jnp.float32),
    )(feat, w1, b1, w2, b2, w3t, b3)


def kernel(compound_diseases, compound_phenotypes,
           compound_subcellular_locations, protein_diseases,
           protein_phenotypes, protein_subcellular_locations,
           disease_table, phenotype_table, sub_table,
           W1, b1, W2, b2, W3, b3):
    cd = compound_diseases.astype(jnp.int32)
    cp = compound_phenotypes.astype(jnp.int32)
    pd = protein_diseases.astype(jnp.int32)
    pp = protein_phenotypes.astype(jnp.int32)
    cs = jnp.pad(compound_subcellular_locations.astype(jnp.int32),
                 ((0, 0), (0, _LSP - _LS)))
    ps = jnp.pad(protein_subcellular_locations.astype(jnp.int32),
                 ((0, 0), (0, _LSP - _LS)))
    sidx = jnp.concatenate([cs, ps], axis=1).reshape(-1)

    feat = _features_sc(cd, cp, sidx, pd, pp,
                        disease_table, phenotype_table, sub_table)
    return _mlp_tc(feat, W1, b1.reshape(1, _H1), W2, b2.reshape(1, _H2),
                   W3.reshape(1, _H2), b3.reshape(1, 1))


# speedup vs baseline: 1.0878x; 1.0160x over previous
import functools

import jax
import jax.numpy as jnp
from jax import lax
from jax.experimental import pallas as pl
from jax.experimental.pallas import tpu as pltpu
from jax.experimental.pallas import tpu_sc as plsc

_B = 16384
_L = 200
_LS = 20
_LSP = 24
_LS2 = 2 * _LSP
_DD, _DP, _DS = 32, 16, 16
_H1, _H2 = 128, 64
_F = 2 * (_DD + _DP + _DS)

_ND, _NP, _NSUB = 13752, 17393, 30
_NC, _NS = 2, 16
_NW = _NC * _NS
_BPW = _B // _NW
_CE = 16
_NCHUNK = _BPW // _CE
_S0, _S1 = 128, 72
_DEPTH = 2
_NSS = (_CE * _LS2) // 128


def _features_sc(cd, cp, sidx, pd, pp, dis_t, phe_t, sub_t):
    mesh = plsc.VectorSubcoreMesh(core_axis_name="c", subcore_axis_name="s")

    @functools.partial(
        pl.kernel,
        mesh=mesh,
        compiler_params=pltpu.CompilerParams(use_tc_tiling_on_sc=False),
        out_type=jax.ShapeDtypeStruct((_B, _F), jnp.float32),
        scratch_types=[
            pltpu.VMEM((2, _CE, _L), jnp.int32),
            pltpu.VMEM((2, _CE, _L), jnp.int32),
            pltpu.VMEM((2, _CE * _LS2), jnp.int32),
            pltpu.VMEM((2, _CE, _L), jnp.int32),
            pltpu.VMEM((2, _CE, _L), jnp.int32),
            pltpu.VMEM((_DEPTH, _L, _DD), jnp.float32),
            pltpu.VMEM((_DEPTH, _L, _DP), jnp.float32),
            pltpu.VMEM((_DEPTH, _L, _DD), jnp.float32),
            pltpu.VMEM((_DEPTH, _L, _DP), jnp.float32),
            pltpu.VMEM((_CE * _LS2, _DS), jnp.float32),
            pltpu.VMEM((_CE, _F), jnp.float32),
            pltpu.VMEM_SHARED((_ND, _DD), jnp.float32),
            pltpu.VMEM_SHARED((_NP, _DP), jnp.float32),
            pltpu.VMEM_SHARED((_NSUB, _DS), jnp.float32),
        ] + [pltpu.SemaphoreType.DMA] * _DEPTH
          + [pltpu.SemaphoreType.DMA] * 2
          + [pltpu.SemaphoreType.DMA],
    )
    def feat_kernel(cd_h, cp_h, sidx_h, pd_h, pp_h, dis_h, phe_h, sub_h,
                    out_h, cd_i, cp_i, sidx_i, pd_i, pp_i,
                    cdb, cpb, pdb, ppb, subr, feat,
                    dis_s, phe_s, sub_s, *sems):
        gsems = sems[:_DEPTH]
        slab_sems = sems[_DEPTH:_DEPTH + 2]
        sub_sem = sems[_DEPTH + 2]
        wid = lax.axis_index("s") * _NC + lax.axis_index("c")
        base = wid * _BPW

        @pl.when(lax.axis_index("s") == 0)
        def _():
            pltpu.sync_copy(dis_h, dis_s)
            pltpu.sync_copy(phe_h, phe_s)
            pltpu.sync_copy(sub_h, sub_s)

        plsc.subcore_barrier()

        def issue(e, si, sem, par):
            for idx_i, buf, tbl in ((cd_i, cdb, dis_s), (cp_i, cpb, phe_s),
                                    (pd_i, pdb, dis_s), (pp_i, ppb, phe_s)):
                pltpu.async_copy(tbl.at[idx_i.at[par, e, pl.ds(0, _S0)]],
                                 buf.at[si, pl.ds(0, _S0)], sem)
                pltpu.async_copy(tbl.at[idx_i.at[par, e, pl.ds(_S0, _S1)]],
                                 buf.at[si, pl.ds(_S0, _S1)], sem)

        def load_slabs(c, par):
            cb = base + c * _CE
            for s_h, dst in ((cd_h, cd_i), (cp_h, cp_i), (pd_h, pd_i),
                             (pp_h, pp_i)):
                pltpu.async_copy(s_h.at[pl.ds(cb, _CE)], dst.at[par],
                                 slab_sems[par])
            pltpu.async_copy(sidx_h.at[pl.ds(cb * _LS2, _CE * _LS2)],
                             sidx_i.at[par], slab_sems[par])

        def wait_slabs(par):
            sem = slab_sems[par]
            pltpu.make_async_copy(cd_h.at[pl.ds(0, _CE)], cd_i.at[par], sem).wait()
            pltpu.make_async_copy(cp_h.at[pl.ds(0, _CE)], cp_i.at[par], sem).wait()
            pltpu.make_async_copy(pd_h.at[pl.ds(0, _CE)], pd_i.at[par], sem).wait()
            pltpu.make_async_copy(pp_h.at[pl.ds(0, _CE)], pp_i.at[par], sem).wait()
            pltpu.make_async_copy(sidx_h.at[pl.ds(0, _CE * _LS2)],
                                  sidx_i.at[par], sem).wait()

        def drain(si, sem):
            pltpu.make_async_copy(dis_h.at[pl.ds(0, _L)], cdb.at[si], sem).wait()
            pltpu.make_async_copy(phe_h.at[pl.ds(0, _L)], cpb.at[si], sem).wait()
            pltpu.make_async_copy(dis_h.at[pl.ds(0, _L)], pdb.at[si], sem).wait()
            pltpu.make_async_copy(phe_h.at[pl.ds(0, _L)], ppb.at[si], sem).wait()

        def reduce_sub(e, off, col):
            accs = [jnp.zeros((16,), jnp.float32) for _ in range(4)]
            rbase = e * _LS2 + off
            for j in range(_LS):
                accs[j % 4] = accs[j % 4] + subr[rbase + j, pl.ds(0, 16)]
            tot = (accs[0] + accs[1]) + (accs[2] + accs[3])
            feat[e, pl.ds(col, 16)] = tot * (1.0 / _LS)

        def reduce_all(si, e):
            nrows = 4
            z = jnp.zeros((16,), jnp.float32)

            @plsc.parallel_loop(0, _L, nrows, unroll=2,
                                carry=tuple(z for _ in range(12)))
            def accs(r0, accs):
                accs = list(accs)
                for i in range(nrows):
                    row = r0 + i
                    ch = i % 2
                    accs[0 + ch] = accs[0 + ch] + cdb[si, row, pl.ds(0, 16)]
                    accs[2 + ch] = accs[2 + ch] + cdb[si, row, pl.ds(16, 16)]
                    accs[4 + ch] = accs[4 + ch] + cpb[si, row, pl.ds(0, 16)]
                    accs[6 + ch] = accs[6 + ch] + pdb[si, row, pl.ds(0, 16)]
                    accs[8 + ch] = accs[8 + ch] + pdb[si, row, pl.ds(16, 16)]
                    accs[10 + ch] = accs[10 + ch] + ppb[si, row, pl.ds(0, 16)]
                return tuple(accs)

            s = 1.0 / _L
            feat[e, pl.ds(0, 16)] = (accs[0] + accs[1]) * s
            feat[e, pl.ds(16, 16)] = (accs[2] + accs[3]) * s
            feat[e, pl.ds(_DD, 16)] = (accs[4] + accs[5]) * s
            feat[e, pl.ds(64, 16)] = (accs[6] + accs[7]) * s
            feat[e, pl.ds(80, 16)] = (accs[8] + accs[9]) * s
            feat[e, pl.ds(64 + _DD, 16)] = (accs[10] + accs[11]) * s
            reduce_sub(e, 0, _DD + _DP)
            reduce_sub(e, _LSP, 64 + _DD + _DP)

        def do_chunk(c, par):
            cbase = base + c * _CE
            wait_slabs(par)

            for k in range(_NSS):
                pltpu.async_copy(
                    sub_s.at[sidx_i.at[par, pl.ds(k * 128, 128)]],
                    subr.at[pl.ds(k * 128, 128)], sub_sem)

            @pl.when(c + 1 < _NCHUNK)
            def _(c=c, par=par):
                load_slabs(c + 1, 1 - par)

            for d in range(_DEPTH - 1):
                issue(d, d, gsems[d], par)

            pltpu.make_async_copy(phe_h.at[pl.ds(0, _CE * _LS2)], subr,
                                  sub_sem).wait()

            def group_body(k, __):
                e = k * _DEPTH
                for d in range(_DEPTH):
                    ed = e + d
                    nxt = ed + _DEPTH - 1
                    tgt = (d + _DEPTH - 1) % _DEPTH
                    if d == 0:
                        issue(nxt, tgt, gsems[tgt], par)
                    else:
                        @pl.when(nxt < _CE)
                        def _(nxt=nxt, tgt=tgt):
                            issue(nxt, tgt, gsems[tgt], par)
                    drain(d, gsems[d])
                    reduce_all(d, ed)
                return 0

            lax.fori_loop(0, _CE // _DEPTH, group_body, 0)
            for t in range((_CE // _DEPTH) * _DEPTH, _CE):
                st = t % _DEPTH
                drain(st, gsems[st])
                reduce_all(st, t)
            pltpu.sync_copy(feat, out_h.at[pl.ds(cbase, _CE)])

        load_slabs(0, 0)

        def chunk_pair(k, _):
            do_chunk(2 * k, 0)
            do_chunk(2 * k + 1, 1)
            return 0

        lax.fori_loop(0, _NCHUNK // 2, chunk_pair, 0)

    return feat_kernel(cd, cp, sidx, pd, pp, dis_t, phe_t, sub_t)


def _mlp_tc(feat, w1, b1, w2, b2, w3t, b3):
    blk = 1024

    def body(x_ref, w1_ref, b1_ref, w2_ref, b2_ref, w3t_ref, b3_ref, o_ref):
        x = x_ref[...]
        h = jnp.dot(x, w1_ref[...], preferred_element_type=jnp.float32)
        h = h + b1_ref[...]
        h = jnp.where(h > 0, h, h * 0.01)
        h = jnp.dot(h, w2_ref[...], preferred_element_type=jnp.float32)
        h = h + b2_ref[...]
        h = jnp.where(h > 0, h, h * 0.01)
        o = jnp.sum(h * w3t_ref[...], axis=1, keepdims=True) + b3_ref[...]
        o_ref[...] = o

    return pl.pallas_call(
        body,
        grid=(_B // blk,),
        in_specs=[
            pl.BlockSpec((blk, _F), lambda i: (i, 0)),
            pl.BlockSpec((_F, _H1), lambda i: (0, 0)),
            pl.BlockSpec((1, _H1), lambda i: (0, 0)),
            pl.BlockSpec((_H1, _H2), lambda i: (0, 0)),
            pl.BlockSpec((1, _H2), lambda i: (0, 0)),
            pl.BlockSpec((1, _H2), lambda i: (0, 0)),
            pl.BlockSpec((1, 1), lambda i: (0, 0)),
        ],
        out_specs=pl.BlockSpec((blk, 1), lambda i: (i, 0)),
        out_shape=jax.ShapeDtypeStruct((_B, 1), jnp.float32),
    )(feat, w1, b1, w2, b2, w3t, b3)


def kernel(compound_diseases, compound_phenotypes,
           compound_subcellular_locations, protein_diseases,
           protein_phenotypes, protein_subcellular_locations,
           disease_table, phenotype_table, sub_table,
           W1, b1, W2, b2, W3, b3):
    cd = compound_diseases.astype(jnp.int32)
    cp = compound_phenotypes.astype(jnp.int32)
    pd = protein_diseases.astype(jnp.int32)
    pp = protein_phenotypes.astype(jnp.int32)
    cs = jnp.pad(compound_subcellular_locations.astype(jnp.int32),
                 ((0, 0), (0, _LSP - _LS)))
    ps = jnp.pad(protein_subcellular_locations.astype(jnp.int32),
                 ((0, 0), (0, _LSP - _LS)))
    sidx = jnp.concatenate([cs, ps], axis=1).reshape(-1)

    feat = _features_sc(cd, cp, sidx, pd, pp,
                        disease_table, phenotype_table, sub_table)
    return _mlp_tc(feat, W1, b1.reshape(1, _H1), W2, b2.reshape(1, _H2),
                   W3.reshape(1, _H2), b3.reshape(1, 1))
